# baseline (speedup 1.0000x reference)
import functools

import jax
import jax.numpy as jnp
from jax import lax
from jax.experimental import pallas as pl
from jax.experimental.pallas import tpu as pltpu
from jax.experimental.pallas import tpu_sc as plsc

N = 10000
D = 128
E = 320000
NS = 16
EPT = E // NS
CHUNK = 125
STEPS = EPT // CHUNK
G = 5
NG = STEPS // G
NQ = NG // 2
SLAB = 632
SLAB_LAST = N - 15 * SLAB
ZR = 96
CW = 16

_mesh = plsc.VectorSubcoreMesh(core_axis_name="c", subcore_axis_name="s")


def _make_agg(with_counts):
    out_type = [jax.ShapeDtypeStruct((2, N, D), jnp.float32)]
    scratch = [
        pltpu.VMEM_SHARED((N, D), jnp.float32),
        pltpu.VMEM((G, CHUNK), jnp.int32),
        pltpu.VMEM((G, CHUNK), jnp.int32),
        pltpu.VMEM((G, CHUNK), jnp.int32),
        pltpu.VMEM((G, CHUNK), jnp.int32),
        pltpu.VMEM((CHUNK, D), jnp.float32),
        pltpu.VMEM((CHUNK, D), jnp.float32),
        pltpu.SemaphoreType.DMA,
        pltpu.SemaphoreType.DMA,
        pltpu.SemaphoreType.DMA,
        pltpu.SemaphoreType.DMA,
        pltpu.SemaphoreType.DMA,
        pltpu.SemaphoreType.DMA,
        pltpu.SemaphoreType.DMA,
    ]
    if with_counts:
        out_type += [jax.ShapeDtypeStruct((N,), jnp.float32),
                     jax.ShapeDtypeStruct((N,), jnp.float32)]
        scratch += [
            pltpu.VMEM_SHARED((N,), jnp.float32),
            pltpu.VMEM((128,), jnp.float32),
            pltpu.VMEM((640,), jnp.float32),
        ]

    def body(x_u, x_i, si_ui, di_ui, si_iu, di_iu, *rest):
        if with_counts:
            (out_sum, cnt_i, cnt_u,
             acc, src_a, dst_a, src_b, dst_b, rows0, rows1,
             sem_g0, sem_g1, sem_s0, sem_s1, sem_c0, sem_c1, sem_i,
             cnt_acc, ones_v, zcnt) = rest
            cnt_outs = (cnt_i, cnt_u)
        else:
            (out_sum,
             acc, src_a, dst_a, src_b, dst_b, rows0, rows1,
             sem_g0, sem_g1, sem_s0, sem_s1, sem_c0, sem_c1, sem_i) = rest
            cnt_outs = cnt_acc = ones_v = zcnt = None
        rows = rows0
        c = lax.axis_index("c")
        s = lax.axis_index("s")
        base = s * SLAB

        @pl.loop(0, ZR)
        def _(i):
            @pl.loop(0, D, step=16)
            def _(j):
                rows[i, pl.ds(j, 16)] = jnp.zeros((16,), jnp.float32)

        if with_counts:
            @pl.loop(0, 128, step=16)
            def _(i):
                ones_v[pl.ds(i, 16)] = jnp.ones((16,), jnp.float32)

            @pl.loop(0, 640, step=16)
            def _(i):
                zcnt[pl.ds(i, 16)] = jnp.zeros((16,), jnp.float32)

        def zero_slab(nrows):
            @pl.loop(0, nrows // ZR)
            def _(i):
                pltpu.sync_copy(rows.at[pl.ds(0, ZR)],
                                acc.at[pl.ds(base + i * ZR, ZR)])
            rem = nrows % ZR
            pltpu.sync_copy(rows.at[pl.ds(0, rem)],
                            acc.at[pl.ds(base + (nrows // ZR) * ZR, rem)])
            if with_counts:
                pltpu.sync_copy(zcnt.at[pl.ds(0, nrows)],
                                cnt_acc.at[pl.ds(base, nrows)])

        @pl.when(s < 15)
        def _():
            zero_slab(SLAB)

        @pl.when(s == 15)
        def _():
            zero_slab(SLAB_LAST)

        plsc.subcore_barrier()
        if not with_counts:
            x_u, x_i = x_u.at[1], x_i.at[0]

        def run_rel(tab, si, di):
            bufs = ((rows0, sem_g0, sem_s0, sem_c0),
                    (rows1, sem_g1, sem_s1, sem_c1))
            pltpu.sync_copy(si.at[s, 0], src_a)
            pltpu.sync_copy(di.at[s, 0], dst_a)
            pltpu.async_copy(tab.at[src_a.at[0]], rows0, sem_g0)

            @pl.loop(0, NQ)
            def _(q):
                for j in range(2 * G):
                    sa, da = (src_a, dst_a) if j < G else (src_b, dst_b)
                    r = j % G
                    buf, sem_g, sem_s, sem_c = bufs[j % 2]
                    nbuf, nsem_g, nsem_s, nsem_c = bufs[1 - j % 2]
                    pltpu.make_async_copy(tab.at[sa.at[r]], buf, sem_g).wait()
                    pltpu.async_copy(buf, acc.at[da.at[r]], sem_s, add=True)
                    if with_counts:
                        pltpu.async_copy(ones_v.at[pl.ds(0, CHUNK)],
                                         cnt_acc.at[da.at[r]], sem_c, add=True)
                    if j == 1:
                        pltpu.async_copy(si.at[s, 2 * q + 1], src_b, sem_i)
                        pltpu.async_copy(di.at[s, 2 * q + 1], dst_b, sem_i)
                    if j == 6:
                        @pl.when(q < NQ - 1)
                        def _():
                            pltpu.async_copy(si.at[s, 2 * q + 2], src_a,
                                             sem_i)
                            pltpu.async_copy(di.at[s, 2 * q + 2], dst_a,
                                             sem_i)

                    def drain_next():
                        pltpu.make_async_copy(
                            nbuf, acc.at[da.at[r]], nsem_s).wait()
                        if with_counts:
                            pltpu.make_async_copy(
                                ones_v.at[pl.ds(0, CHUNK)],
                                cnt_acc.at[da.at[r]], nsem_c).wait()

                    if j == 0:
                        @pl.when(q > 0)
                        def _():
                            drain_next()
                        pltpu.async_copy(tab.at[sa.at[1]], nbuf, nsem_g)
                    elif j < G - 1:
                        drain_next()
                        pltpu.async_copy(tab.at[sa.at[r + 1]], nbuf, nsem_g)
                    elif j == G - 1:
                        drain_next()
                        pltpu.make_async_copy(si.at[s, 2 * q + 1], src_b,
                                              sem_i).wait()
                        pltpu.make_async_copy(di.at[s, 2 * q + 1], dst_b,
                                              sem_i).wait()
                        pltpu.async_copy(tab.at[src_b.at[0]], nbuf, nsem_g)
                    elif j < 2 * G - 1:
                        drain_next()
                        pltpu.async_copy(tab.at[sa.at[r + 1]], nbuf, nsem_g)
                    else:
                        @pl.when(q < NQ - 1)
                        def _():
                            drain_next()
                            pltpu.make_async_copy(si.at[s, 2 * q + 2], src_a,
                                                  sem_i).wait()
                            pltpu.make_async_copy(di.at[s, 2 * q + 2], dst_a,
                                                  sem_i).wait()
                            pltpu.async_copy(tab.at[src_a.at[0]], nbuf,
                                             nsem_g)

            pltpu.make_async_copy(rows0, acc.at[dst_b.at[G - 2]],
                                  sem_s0).wait()
            pltpu.make_async_copy(rows1, acc.at[dst_b.at[G - 1]],
                                  sem_s1).wait()
            if with_counts:
                pltpu.make_async_copy(ones_v.at[pl.ds(0, CHUNK)],
                                      cnt_acc.at[dst_b.at[G - 2]],
                                      sem_c0).wait()
                pltpu.make_async_copy(ones_v.at[pl.ds(0, CHUNK)],
                                      cnt_acc.at[dst_b.at[G - 1]],
                                      sem_c1).wait()

        @pl.when(c == 0)
        def _():
            run_rel(x_u, si_ui, di_ui)

        @pl.when(c == 1)
        def _():
            run_rel(x_i, si_iu, di_iu)

        plsc.subcore_barrier()

        def dump(k, nrows):
            pltpu.sync_copy(acc.at[pl.ds(base, nrows)],
                            out_sum.at[k, pl.ds(base, nrows)])
            if with_counts:
                pltpu.sync_copy(cnt_acc.at[pl.ds(base, nrows)],
                                zcnt.at[pl.ds(0, nrows)])
                pltpu.sync_copy(zcnt.at[pl.ds(0, nrows)],
                                cnt_outs[k].at[pl.ds(base, nrows)])

        for k in (0, 1):
            for last in (False, True):
                @pl.when((c == k) & ((s == 15) if last else (s < 15)))
                def _(k=k, last=last):
                    dump(k, SLAB_LAST if last else SLAB)

    return pl.kernel(body, out_type=out_type, mesh=_mesh,
                     scratch_types=scratch, name=f"sage_agg_{int(with_counts)}")


_agg_with_counts = _make_agg(True)
_agg_no_counts = _make_agg(False)

BR = 1000


def _dense_body(s_ref, c_ref, x_ref, wl_ref, b_ref, wr_ref, o_ref):
    cnt = c_ref[0]
    agg = s_ref[0] * (1.0 / jnp.maximum(cnt, 1.0))
    acc = jnp.dot(agg, wl_ref[0], preferred_element_type=jnp.float32,
                  precision=lax.Precision.HIGHEST)
    acc = acc + b_ref[0]
    acc = acc + jnp.dot(x_ref[0], wr_ref[0],
                        preferred_element_type=jnp.float32,
                        precision=lax.Precision.HIGHEST)
    o_ref[0] = jnp.maximum(acc, 0.0)


_dense_call = pl.pallas_call(
    _dense_body,
    grid=(2, N // BR),
    in_specs=[
        pl.BlockSpec((1, BR, D), lambda k, i: (k, i, 0)),
        pl.BlockSpec((1, BR, 1), lambda k, i: (k, i, 0)),
        pl.BlockSpec((1, BR, D), lambda k, i: (k, i, 0)),
        pl.BlockSpec((1, D, D), lambda k, i: (k, 0, 0)),
        pl.BlockSpec((1, 1, D), lambda k, i: (k, 0, 0)),
        pl.BlockSpec((1, D, D), lambda k, i: (k, 0, 0)),
    ],
    out_specs=pl.BlockSpec((1, BR, D), lambda k, i: (k, i, 0)),
    out_shape=jax.ShapeDtypeStruct((2, N, D), jnp.float32),
)


def _dense(S, C, X, Wl, B, Wr):
    return _dense_call(S, C.reshape(2, N, 1), X, Wl, B, Wr)


def kernel(x_user, x_item, edge_index_ui, edge_index_iu,
           W1_ui_l, b1_ui_l, W1_ui_r, W1_iu_l, b1_iu_l, W1_iu_r,
           W2_ui_l, b2_ui_l, W2_ui_r, W2_iu_l, b2_iu_l, W2_iu_r):
    si_ui = edge_index_ui[0].reshape(NS, NG, G, CHUNK)
    di_ui = edge_index_ui[1].reshape(NS, NG, G, CHUNK)
    si_iu = edge_index_iu[0].reshape(NS, NG, G, CHUNK)
    di_iu = edge_index_iu[1].reshape(NS, NG, G, CHUNK)

    Wl1 = jnp.stack([W1_ui_l.T, W1_iu_l.T])
    Wr1 = jnp.stack([W1_ui_r.T, W1_iu_r.T])
    B1 = jnp.stack([b1_ui_l, b1_iu_l]).reshape(2, 1, D)
    Wl2 = jnp.stack([W2_ui_l.T, W2_iu_l.T])
    Wr2 = jnp.stack([W2_ui_r.T, W2_iu_r.T])
    B2 = jnp.stack([b2_ui_l, b2_iu_l]).reshape(2, 1, D)
    X1 = jnp.stack([x_item, x_user])

    S1, C_i, C_u = _agg_with_counts(x_user, x_item, si_ui, di_ui, si_iu,
                                    di_iu)
    C1 = jnp.stack([C_i, C_u])
    O1 = _dense(S1, C1, X1, Wl1, B1, Wr1)
    (S2,) = _agg_no_counts(O1, O1, si_ui, di_ui, si_iu, di_iu)
    O2 = _dense(S2, C1, O1, Wl2, B2, Wr2)
    return (O2[1], O2[0])


# speedup vs baseline: 1.0526x; 1.0526x over previous
import functools

import jax
import jax.numpy as jnp
from jax import lax
from jax.experimental import pallas as pl
from jax.experimental.pallas import tpu as pltpu
from jax.experimental.pallas import tpu_sc as plsc

N = 10000
D = 128
E = 320000
NS = 16
EPT = E // NS
CHUNK = 125
STEPS = EPT // CHUNK
G = 5
NG = STEPS // G
NQ = NG // 2
SLAB = 632
SLAB_LAST = N - 15 * SLAB
ZR = 96
CW = 16

_mesh = plsc.VectorSubcoreMesh(core_axis_name="c", subcore_axis_name="s")


def _make_agg(with_counts):
    out_type = [jax.ShapeDtypeStruct((2, N, D), jnp.float32)]
    scratch = [
        pltpu.VMEM_SHARED((N, D), jnp.float32),
        pltpu.VMEM((G, CHUNK), jnp.int32),
        pltpu.VMEM((G, CHUNK), jnp.int32),
        pltpu.VMEM((G, CHUNK), jnp.int32),
        pltpu.VMEM((G, CHUNK), jnp.int32),
        pltpu.VMEM((CHUNK, D), jnp.float32),
        pltpu.VMEM((CHUNK, D), jnp.float32),
        pltpu.SemaphoreType.DMA,
        pltpu.SemaphoreType.DMA,
        pltpu.SemaphoreType.DMA,
        pltpu.SemaphoreType.DMA,
        pltpu.SemaphoreType.DMA,
        pltpu.SemaphoreType.DMA,
        pltpu.SemaphoreType.DMA,
    ]
    if with_counts:
        out_type += [jax.ShapeDtypeStruct((N,), jnp.float32),
                     jax.ShapeDtypeStruct((N,), jnp.float32)]
        scratch += [
            pltpu.VMEM_SHARED((N,), jnp.float32),
            pltpu.VMEM((128,), jnp.float32),
            pltpu.VMEM((640,), jnp.float32),
        ]

    def body(x_u, x_i, si_ui, di_ui, si_iu, di_iu, *rest):
        if with_counts:
            (out_sum, cnt_i, cnt_u,
             acc, src_a, dst_a, src_b, dst_b, rows0, rows1,
             sem_g0, sem_g1, sem_s0, sem_s1, sem_c0, sem_c1, sem_i,
             cnt_acc, ones_v, zcnt) = rest
            cnt_outs = (cnt_i, cnt_u)
        else:
            (out_sum,
             acc, src_a, dst_a, src_b, dst_b, rows0, rows1,
             sem_g0, sem_g1, sem_s0, sem_s1, sem_c0, sem_c1, sem_i) = rest
            cnt_outs = cnt_acc = ones_v = zcnt = None
        rows = rows0
        c = lax.axis_index("c")
        s = lax.axis_index("s")
        base = s * SLAB

        @pl.loop(0, ZR)
        def _(i):
            @pl.loop(0, D, step=16)
            def _(j):
                rows[i, pl.ds(j, 16)] = jnp.zeros((16,), jnp.float32)

        if with_counts:
            @pl.loop(0, 128, step=16)
            def _(i):
                ones_v[pl.ds(i, 16)] = jnp.ones((16,), jnp.float32)

            @pl.loop(0, 640, step=16)
            def _(i):
                zcnt[pl.ds(i, 16)] = jnp.zeros((16,), jnp.float32)

        def zero_slab(nrows):
            @pl.loop(0, nrows // ZR)
            def _(i):
                pltpu.sync_copy(rows.at[pl.ds(0, ZR)],
                                acc.at[pl.ds(base + i * ZR, ZR)])
            rem = nrows % ZR
            pltpu.sync_copy(rows.at[pl.ds(0, rem)],
                            acc.at[pl.ds(base + (nrows // ZR) * ZR, rem)])
            if with_counts:
                pltpu.sync_copy(zcnt.at[pl.ds(0, nrows)],
                                cnt_acc.at[pl.ds(base, nrows)])

        @pl.when(s < 15)
        def _():
            zero_slab(SLAB)

        @pl.when(s == 15)
        def _():
            zero_slab(SLAB_LAST)

        plsc.subcore_barrier()
        if not with_counts:
            x_u, x_i = x_u.at[1], x_i.at[0]

        def run_rel(tab, si, di):
            bufs = ((rows0, sem_g0, sem_s0, sem_c0),
                    (rows1, sem_g1, sem_s1, sem_c1))
            pltpu.sync_copy(si.at[s, 0], src_a)
            pltpu.sync_copy(di.at[s, 0], dst_a)
            pltpu.async_copy(tab.at[src_a.at[0]], rows0, sem_g0)

            @pl.loop(0, NQ)
            def _(q):
                for j in range(2 * G):
                    sa, da = (src_a, dst_a) if j < G else (src_b, dst_b)
                    r = j % G
                    buf, sem_g, sem_s, sem_c = bufs[j % 2]
                    nbuf, nsem_g, nsem_s, nsem_c = bufs[1 - j % 2]
                    pltpu.make_async_copy(tab.at[sa.at[r]], buf, sem_g).wait()
                    pltpu.async_copy(buf, acc.at[da.at[r]], sem_s, add=True)
                    if with_counts:
                        pltpu.async_copy(ones_v.at[pl.ds(0, CHUNK)],
                                         cnt_acc.at[da.at[r]], sem_c, add=True)
                    if j == 1:
                        pltpu.async_copy(si.at[s, 2 * q + 1], src_b, sem_i)
                        pltpu.async_copy(di.at[s, 2 * q + 1], dst_b, sem_i)
                    if j == 6:
                        @pl.when(q < NQ - 1)
                        def _():
                            pltpu.async_copy(si.at[s, 2 * q + 2], src_a,
                                             sem_i)
                            pltpu.async_copy(di.at[s, 2 * q + 2], dst_a,
                                             sem_i)

                    def drain_next():
                        pltpu.make_async_copy(
                            nbuf, acc.at[da.at[r]], nsem_s).wait()
                        if with_counts:
                            pltpu.make_async_copy(
                                ones_v.at[pl.ds(0, CHUNK)],
                                cnt_acc.at[da.at[r]], nsem_c).wait()

                    if j == 0:
                        @pl.when(q > 0)
                        def _():
                            drain_next()
                        pltpu.async_copy(tab.at[sa.at[1]], nbuf, nsem_g)
                    elif j < G - 1:
                        drain_next()
                        pltpu.async_copy(tab.at[sa.at[r + 1]], nbuf, nsem_g)
                    elif j == G - 1:
                        drain_next()
                        pltpu.make_async_copy(si.at[s, 2 * q + 1], src_b,
                                              sem_i).wait()
                        pltpu.make_async_copy(di.at[s, 2 * q + 1], dst_b,
                                              sem_i).wait()
                        pltpu.async_copy(tab.at[src_b.at[0]], nbuf, nsem_g)
                    elif j < 2 * G - 1:
                        drain_next()
                        pltpu.async_copy(tab.at[sa.at[r + 1]], nbuf, nsem_g)
                    else:
                        @pl.when(q < NQ - 1)
                        def _():
                            drain_next()
                            pltpu.make_async_copy(si.at[s, 2 * q + 2], src_a,
                                                  sem_i).wait()
                            pltpu.make_async_copy(di.at[s, 2 * q + 2], dst_a,
                                                  sem_i).wait()
                            pltpu.async_copy(tab.at[src_a.at[0]], nbuf,
                                             nsem_g)

            pltpu.make_async_copy(rows0, acc.at[dst_b.at[G - 2]],
                                  sem_s0).wait()
            pltpu.make_async_copy(rows1, acc.at[dst_b.at[G - 1]],
                                  sem_s1).wait()
            if with_counts:
                pltpu.make_async_copy(ones_v.at[pl.ds(0, CHUNK)],
                                      cnt_acc.at[dst_b.at[G - 2]],
                                      sem_c0).wait()
                pltpu.make_async_copy(ones_v.at[pl.ds(0, CHUNK)],
                                      cnt_acc.at[dst_b.at[G - 1]],
                                      sem_c1).wait()

        @pl.when(c == 0)
        def _():
            run_rel(x_u, si_ui, di_ui)

        @pl.when(c == 1)
        def _():
            run_rel(x_i, si_iu, di_iu)

        plsc.subcore_barrier()

        def dump(k, nrows):
            pltpu.sync_copy(acc.at[pl.ds(base, nrows)],
                            out_sum.at[k, pl.ds(base, nrows)])
            if with_counts:
                pltpu.sync_copy(cnt_acc.at[pl.ds(base, nrows)],
                                zcnt.at[pl.ds(0, nrows)])
                pltpu.sync_copy(zcnt.at[pl.ds(0, nrows)],
                                cnt_outs[k].at[pl.ds(base, nrows)])

        for k in (0, 1):
            for last in (False, True):
                @pl.when((c == k) & ((s == 15) if last else (s < 15)))
                def _(k=k, last=last):
                    dump(k, SLAB_LAST if last else SLAB)

    return pl.kernel(body, out_type=out_type, mesh=_mesh,
                     scratch_types=scratch, name=f"sage_agg_{int(with_counts)}")


_agg_with_counts = _make_agg(True)
_agg_no_counts = _make_agg(False)

BR = 2000


def _dense_body(s_ref, c_ref, x_ref, wl_ref, b_ref, wr_ref, o_ref):
    cnt = c_ref[0]
    agg = s_ref[0] * (1.0 / jnp.maximum(cnt, 1.0))
    acc = jnp.dot(agg, wl_ref[0], preferred_element_type=jnp.float32,
                  precision=lax.Precision.HIGHEST)
    acc = acc + b_ref[0]
    acc = acc + jnp.dot(x_ref[0], wr_ref[0],
                        preferred_element_type=jnp.float32,
                        precision=lax.Precision.HIGHEST)
    o_ref[0] = jnp.maximum(acc, 0.0)


_dense_call = pl.pallas_call(
    _dense_body,
    grid=(2, N // BR),
    in_specs=[
        pl.BlockSpec((1, BR, D), lambda k, i: (k, i, 0)),
        pl.BlockSpec((1, BR, 1), lambda k, i: (k, i, 0)),
        pl.BlockSpec((1, BR, D), lambda k, i: (k, i, 0)),
        pl.BlockSpec((1, D, D), lambda k, i: (k, 0, 0)),
        pl.BlockSpec((1, 1, D), lambda k, i: (k, 0, 0)),
        pl.BlockSpec((1, D, D), lambda k, i: (k, 0, 0)),
    ],
    out_specs=pl.BlockSpec((1, BR, D), lambda k, i: (k, i, 0)),
    out_shape=jax.ShapeDtypeStruct((2, N, D), jnp.float32),
)


def _dense(S, C, X, Wl, B, Wr):
    return _dense_call(S, C.reshape(2, N, 1), X, Wl, B, Wr)


def kernel(x_user, x_item, edge_index_ui, edge_index_iu,
           W1_ui_l, b1_ui_l, W1_ui_r, W1_iu_l, b1_iu_l, W1_iu_r,
           W2_ui_l, b2_ui_l, W2_ui_r, W2_iu_l, b2_iu_l, W2_iu_r):
    si_ui = edge_index_ui[0].reshape(NS, NG, G, CHUNK)
    di_ui = edge_index_ui[1].reshape(NS, NG, G, CHUNK)
    si_iu = edge_index_iu[0].reshape(NS, NG, G, CHUNK)
    di_iu = edge_index_iu[1].reshape(NS, NG, G, CHUNK)

    Wl1 = jnp.stack([W1_ui_l.T, W1_iu_l.T])
    Wr1 = jnp.stack([W1_ui_r.T, W1_iu_r.T])
    B1 = jnp.stack([b1_ui_l, b1_iu_l]).reshape(2, 1, D)
    Wl2 = jnp.stack([W2_ui_l.T, W2_iu_l.T])
    Wr2 = jnp.stack([W2_ui_r.T, W2_iu_r.T])
    B2 = jnp.stack([b2_ui_l, b2_iu_l]).reshape(2, 1, D)
    X1 = jnp.stack([x_item, x_user])

    S1, C_i, C_u = _agg_with_counts(x_user, x_item, si_ui, di_ui, si_iu,
                                    di_iu)
    C1 = jnp.stack([C_i, C_u])
    O1 = _dense(S1, C1, X1, Wl1, B1, Wr1)
    (S2,) = _agg_no_counts(O1, O1, si_ui, di_ui, si_iu, di_iu)
    O2 = _dense(S2, C1, O1, Wl2, B2, Wr2)
    return (O2[1], O2[0])
